# baseline (speedup 1.0000x reference)
import functools

import jax
import jax.numpy as jnp
from jax import lax
from jax.experimental import pallas as pl
from jax.experimental.pallas import tpu as pltpu
from jax.experimental.pallas import tpu_sc as plsc

NC = 2
NS = 16
LANES = 16


def _mm_body(x_ref, w_ref, h_ref):
    h_ref[0] = jnp.dot(x_ref[...], w_ref[0], preferred_element_type=jnp.float32)


def _matmul(x, W, bn):
    N, D_in = x.shape
    K, _, D_out = W.shape
    nb = N // bn
    return pl.pallas_call(
        _mm_body,
        grid=(nb, K),
        in_specs=[
            pl.BlockSpec((bn, D_in), lambda n, k: (n, 0)),
            pl.BlockSpec((1, D_in, D_out), lambda n, k: (k, 0, 0)),
        ],
        out_specs=pl.BlockSpec((1, bn, D_out), lambda n, k: (k, n, 0)),
        out_shape=jax.ShapeDtypeStruct((K, N, D_out), jnp.float32),
    )(x, W)


def _make_sc_scatter(K, N, D, E):
    Dh = D // 2
    ept = E // NS
    C = 80
    NBUF = 5
    nchunk = ept // C
    npair = nchunk // NBUF
    n_pad = -(-N // (NS * 8)) * (NS * 8)
    rpt = n_pad // NS
    mesh = plsc.VectorSubcoreMesh(core_axis_name="c", subcore_axis_name="s")

    @functools.partial(
        pl.kernel,
        out_type=jax.ShapeDtypeStruct((NC * K * n_pad, Dh), jnp.float32),
        mesh=mesh,
        scratch_types=[
            [pltpu.VMEM((C,), jnp.int32) for _ in range(NBUF)],
            [pltpu.VMEM((C,), jnp.int32) for _ in range(NBUF)],
            [pltpu.VMEM((C, Dh), jnp.float32) for _ in range(NBUF)],
            pltpu.VMEM((rpt, Dh), jnp.float32),
            pltpu.VMEM_SHARED((n_pad, Dh), jnp.float32),
            [pltpu.SemaphoreType.DMA for _ in range(NBUF)],
            [pltpu.SemaphoreType.DMA for _ in range(NBUF)],
            [pltpu.SemaphoreType.DMA for _ in range(NBUF)],
            pltpu.SemaphoreType.DMA,
        ],
        compiler_params=pltpu.CompilerParams(use_tc_tiling_on_sc=False),
    )
    def sc_scatter(h_hbm, src_hbm, dst_hbm, out_hbm, sidx, didx, rows, zbuf, acc,
                   isem, dsem, gsem, ssem):
        c = lax.axis_index("c")
        s = lax.axis_index("s")
        zv = jnp.zeros((LANES,), jnp.float32)

        def zrow(r, carry):
            for q in range(Dh // LANES):
                zbuf[r, pl.ds(q * LANES, LANES)] = zv
            return carry

        lax.fori_loop(0, rpt, zrow, 0)
        r0 = s * rpt
        for i in range(K):
            pltpu.sync_copy(zbuf, acc.at[pl.ds(r0, rpt)])
            plsc.subcore_barrier()

            def burst(t, carry):
                base = t * NBUF
                icps = [
                    pltpu.async_copy(src_hbm.at[c, i, s, base + b], sidx[b], isem[b])
                    for b in range(NBUF)
                ]
                dcps = [
                    pltpu.async_copy(dst_hbm.at[i, s, base + b], didx[b], dsem[b])
                    for b in range(NBUF)
                ]
                gcps = []
                for b in range(NBUF):
                    icps[b].wait()
                    gcps.append(pltpu.async_copy(h_hbm.at[sidx[b]], rows[b], gsem[b]))
                for b in range(NBUF):
                    gcps[b].wait()
                    dcps[b].wait()
                    pltpu.async_copy(rows[b], acc.at[didx[b]], ssem, add=True).wait()
                return carry

            lax.fori_loop(0, npair, burst, 0)
            plsc.subcore_barrier()
            obase = (c * K + i) * n_pad + r0
            pltpu.sync_copy(acc.at[pl.ds(r0, rpt)], out_hbm.at[pl.ds(obase, rpt)])

    return sc_scatter, n_pad


def _fc_body(K, p_ref, w1_ref, b1_ref, w2_ref, b2_ref, w3_ref, b3_ref, o_ref):
    p = p_ref[...]
    halves = []
    for h in range(2):
        a = [jnp.maximum(p[h, i], 0.0) for i in range(K)]
        b = [
            jnp.maximum(
                sum(a[i] * w1_ref[i, j] for i in range(K)) + b1_ref[j], 0.0
            )
            for j in range(K)
        ]
        c = [
            jnp.maximum(
                sum(b[i] * w2_ref[i, j] for i in range(K)) + b2_ref[j], 0.0
            )
            for j in range(K)
        ]
        halves.append(sum(c[i] * w3_ref[i, 0] for i in range(K)) + b3_ref[0])
    o_ref[...] = jnp.concatenate(halves, axis=-1)


def _fc(partial, fc1_w, fc1_b, fc2_w, fc2_b, fc3_w, fc3_b, N, bn):
    _, K, _, Dh = partial.shape
    nb = N // bn
    smem = pl.BlockSpec(memory_space=pltpu.SMEM)
    return pl.pallas_call(
        functools.partial(_fc_body, K),
        grid=(nb,),
        in_specs=[
            pl.BlockSpec((2, K, bn, Dh), lambda n: (0, 0, n, 0)),
            smem, smem, smem, smem, smem, smem,
        ],
        out_specs=pl.BlockSpec((bn, 2 * Dh), lambda n: (n, 0)),
        out_shape=jax.ShapeDtypeStruct((N, 2 * Dh), jnp.float32),
    )(partial, fc1_w, fc1_b, fc2_w, fc2_b, fc3_w, fc3_b)


def kernel(x, edge_index, W, fc1_w, fc1_b, fc2_w, fc2_b, fc3_w, fc3_b):
    N, D_in = x.shape
    K, _, D_out = W.shape
    E = edge_index.shape[2]
    h = _matmul(x, W, bn=2000)
    h_half = h.reshape(K * N * 2, D_out // 2)
    nchunk = E // (NS * 80)
    src = edge_index[:, 0, :]
    src_adj = (
        2 * (src[None] + (jnp.arange(K, dtype=jnp.int32) * N)[None, :, None])
        + jnp.arange(NC, dtype=jnp.int32)[:, None, None]
    ).reshape(NC, K, NS, nchunk, 80)
    dst = edge_index[:, 1, :].reshape(K, NS, nchunk, 80)
    sc, n_pad = _make_sc_scatter(K, N, D_out, E)
    partial = sc(h_half, src_adj, dst)
    partial = partial.reshape(2, K, n_pad, D_out // 2)
    return _fc(partial, fc1_w, fc1_b, fc2_w, fc2_b, fc3_w, fc3_b, N, bn=2000)


# speedup vs baseline: 1.1812x; 1.1812x over previous
import functools

import jax
import jax.numpy as jnp
from jax import lax
from jax.experimental import pallas as pl
from jax.experimental.pallas import tpu as pltpu
from jax.experimental.pallas import tpu_sc as plsc

NC = 2
NS = 16
LANES = 16


def _mm_body(x_ref, w_ref, h_ref):
    h_ref[0] = jnp.dot(x_ref[...], w_ref[0], preferred_element_type=jnp.float32)


def _matmul(x, W, bn):
    N, D_in = x.shape
    K, _, D_out = W.shape
    nb = N // bn
    return pl.pallas_call(
        _mm_body,
        grid=(nb, K),
        in_specs=[
            pl.BlockSpec((bn, D_in), lambda n, k: (n, 0)),
            pl.BlockSpec((1, D_in, D_out), lambda n, k: (k, 0, 0)),
        ],
        out_specs=pl.BlockSpec((1, bn, D_out), lambda n, k: (k, n, 0)),
        out_shape=jax.ShapeDtypeStruct((K, N, D_out), jnp.float32),
    )(x, W)


def _make_sc_scatter(K, N, D, E):
    Dh = D // 2
    ept = E // NS
    C = 128
    NBUF = 5
    nfull = ept // C
    rem = ept - nfull * C
    nburst = nfull // NBUF
    nleft = nfull - nburst * NBUF
    n_pad = -(-N // (NS * 8)) * (NS * 8)
    rpt = n_pad // NS
    zrows = rpt // 2
    mesh = plsc.VectorSubcoreMesh(core_axis_name="c", subcore_axis_name="s")

    @functools.partial(
        pl.kernel,
        out_type=jax.ShapeDtypeStruct((NC * K * n_pad, Dh), jnp.float32),
        mesh=mesh,
        scratch_types=[
            [pltpu.VMEM((C,), jnp.int32) for _ in range(NBUF)],
            [pltpu.VMEM((C,), jnp.int32) for _ in range(NBUF)],
            [pltpu.VMEM((C, Dh), jnp.float32) for _ in range(NBUF)],
            pltpu.VMEM((max(rem, 8), Dh), jnp.float32),
            pltpu.VMEM((max(rem, 8),), jnp.int32),
            pltpu.VMEM((max(rem, 8),), jnp.int32),
            pltpu.VMEM((zrows, Dh), jnp.float32),
            pltpu.VMEM_SHARED((n_pad, Dh), jnp.float32),
            [pltpu.SemaphoreType.DMA for _ in range(NBUF)],
            [pltpu.SemaphoreType.DMA for _ in range(NBUF)],
            [pltpu.SemaphoreType.DMA for _ in range(NBUF)],
            pltpu.SemaphoreType.DMA,
        ],
        compiler_params=pltpu.CompilerParams(use_tc_tiling_on_sc=False),
    )
    def sc_scatter(h_hbm, src_hbm, dst_hbm, out_hbm, sidx, didx, rows, rows_r,
                   sidx_r, didx_r, zbuf, acc, isem, dsem, gsem, ssem):
        c = lax.axis_index("c")
        s = lax.axis_index("s")
        zv = jnp.zeros((LANES,), jnp.float32)

        def zrow(r, carry):
            for q in range(Dh // LANES):
                zbuf[r, pl.ds(q * LANES, LANES)] = zv
            return carry

        lax.fori_loop(0, zrows, zrow, 0)
        r0 = s * rpt
        eoff = s * ept
        for i in range(K):
            for j in range(rpt // zrows):
                pltpu.sync_copy(zbuf, acc.at[pl.ds(r0 + j * zrows, zrows)])
            plsc.subcore_barrier()

            def do_chunks(first_chunk, cnt, sb, db, rb, ism, dsm, gsm):
                off = eoff + first_chunk * C
                icp = pltpu.async_copy(
                    src_hbm.at[c, i, pl.ds(off, cnt)], sb, ism)
                dcp = pltpu.async_copy(
                    dst_hbm.at[i, pl.ds(off, cnt)], db, dsm)
                return icp, dcp

            def burst(t, carry):
                base = t * NBUF
                cps = [
                    do_chunks(base + b, C, sidx[b], didx[b], rows[b],
                              isem[b], dsem[b], gsem[b])
                    for b in range(NBUF)
                ]
                gcps = []
                for b in range(NBUF):
                    cps[b][0].wait()
                    gcps.append(pltpu.async_copy(h_hbm.at[sidx[b]], rows[b], gsem[b]))
                for b in range(NBUF):
                    gcps[b].wait()
                    cps[b][1].wait()
                    pltpu.async_copy(rows[b], acc.at[didx[b]], ssem, add=True).wait()
                return carry

            lax.fori_loop(0, nburst, burst, 0)

            eps = []
            for b in range(nleft):
                eps.append((do_chunks(nburst * NBUF + b, C, sidx[b], didx[b],
                                      rows[b], isem[b], dsem[b], gsem[b]),
                            sidx[b], didx[b], rows[b], gsem[b]))
            if rem:
                off = eoff + nfull * C
                icp = pltpu.async_copy(
                    src_hbm.at[c, i, pl.ds(off, rem)], sidx_r, isem[NBUF - 1])
                dcp = pltpu.async_copy(
                    dst_hbm.at[i, pl.ds(off, rem)], didx_r, dsem[NBUF - 1])
                eps.append(((icp, dcp), sidx_r, didx_r, rows_r, gsem[NBUF - 1]))
            gcps = []
            for (icp, dcp), sb, db, rb, gsm in eps:
                icp.wait()
                gcps.append((pltpu.async_copy(h_hbm.at[sb], rb, gsm), dcp, db, rb))
            for gcp, dcp, db, rb in gcps:
                gcp.wait()
                dcp.wait()
                pltpu.async_copy(rb, acc.at[db], ssem, add=True).wait()

            plsc.subcore_barrier()
            obase = (c * K + i) * n_pad + r0
            pltpu.sync_copy(acc.at[pl.ds(r0, rpt)], out_hbm.at[pl.ds(obase, rpt)])

    return sc_scatter, n_pad


def _fc_body(K, p_ref, w1_ref, b1_ref, w2_ref, b2_ref, w3_ref, b3_ref, o_ref):
    p = p_ref[...]
    halves = []
    for h in range(2):
        a = [jnp.maximum(p[h, i], 0.0) for i in range(K)]
        b = [
            jnp.maximum(
                sum(a[i] * w1_ref[i, j] for i in range(K)) + b1_ref[j], 0.0
            )
            for j in range(K)
        ]
        c = [
            jnp.maximum(
                sum(b[i] * w2_ref[i, j] for i in range(K)) + b2_ref[j], 0.0
            )
            for j in range(K)
        ]
        halves.append(sum(c[i] * w3_ref[i, 0] for i in range(K)) + b3_ref[0])
    o_ref[...] = jnp.concatenate(halves, axis=-1)


def _fc(partial, fc1_w, fc1_b, fc2_w, fc2_b, fc3_w, fc3_b, N, bn):
    _, K, _, Dh = partial.shape
    nb = N // bn
    smem = pl.BlockSpec(memory_space=pltpu.SMEM)
    return pl.pallas_call(
        functools.partial(_fc_body, K),
        grid=(nb,),
        in_specs=[
            pl.BlockSpec((2, K, bn, Dh), lambda n: (0, 0, n, 0)),
            smem, smem, smem, smem, smem, smem,
        ],
        out_specs=pl.BlockSpec((bn, 2 * Dh), lambda n: (n, 0)),
        out_shape=jax.ShapeDtypeStruct((N, 2 * Dh), jnp.float32),
    )(partial, fc1_w, fc1_b, fc2_w, fc2_b, fc3_w, fc3_b)


def kernel(x, edge_index, W, fc1_w, fc1_b, fc2_w, fc2_b, fc3_w, fc3_b):
    N, D_in = x.shape
    K, _, D_out = W.shape
    E = edge_index.shape[2]
    h = _matmul(x, W, bn=2000)
    h_half = h.reshape(K * N * 2, D_out // 2)
    src = edge_index[:, 0, :]
    src_adj = (
        2 * (src[None] + (jnp.arange(K, dtype=jnp.int32) * N)[None, :, None])
        + jnp.arange(NC, dtype=jnp.int32)[:, None, None]
    )
    dst = edge_index[:, 1, :]
    sc, n_pad = _make_sc_scatter(K, N, D_out, E)
    partial = sc(h_half, src_adj, dst)
    partial = partial.reshape(2, K, n_pad, D_out // 2)
    return _fc(partial, fc1_w, fc1_b, fc2_w, fc2_b, fc3_w, fc3_b, N, bn=2000)


# speedup vs baseline: 1.2255x; 1.0375x over previous
import functools

import jax
import jax.numpy as jnp
from jax import lax
from jax.experimental import pallas as pl
from jax.experimental.pallas import tpu as pltpu
from jax.experimental.pallas import tpu_sc as plsc

NC = 2
NS = 16
LANES = 16


def _mm_body(x_ref, w_ref, h_ref):
    h_ref[0] = jnp.dot(x_ref[...], w_ref[0], preferred_element_type=jnp.float32)


def _matmul(x, W, bn):
    N, D_in = x.shape
    K, _, D_out = W.shape
    nb = N // bn
    return pl.pallas_call(
        _mm_body,
        grid=(nb, K),
        in_specs=[
            pl.BlockSpec((bn, D_in), lambda n, k: (n, 0)),
            pl.BlockSpec((1, D_in, D_out), lambda n, k: (k, 0, 0)),
        ],
        out_specs=pl.BlockSpec((1, bn, D_out), lambda n, k: (k, n, 0)),
        out_shape=jax.ShapeDtypeStruct((K, N, D_out), jnp.float32),
    )(x, W)


def _make_sc_scatter(K, N, D, E):
    Dh = D // 2
    ept = E // NS
    C = 128
    NBUF = 5
    nfull = ept // C
    rem = ept - nfull * C
    nburst = nfull // NBUF
    nleft = nfull - nburst * NBUF
    n_pad = -(-N // (NS * 8)) * (NS * 8)
    rpt = n_pad // NS
    zrows = rpt // 2
    mesh = plsc.VectorSubcoreMesh(core_axis_name="c", subcore_axis_name="s")

    @functools.partial(
        pl.kernel,
        out_type=jax.ShapeDtypeStruct((NC * K * n_pad, Dh), jnp.float32),
        mesh=mesh,
        scratch_types=[
            [pltpu.VMEM((C,), jnp.int32) for _ in range(NBUF)],
            [pltpu.VMEM((C,), jnp.int32) for _ in range(NBUF)],
            [pltpu.VMEM((C, Dh), jnp.float32) for _ in range(NBUF)],
            pltpu.VMEM((max(rem, 8), Dh), jnp.float32),
            pltpu.VMEM((max(rem, 8),), jnp.int32),
            pltpu.VMEM((max(rem, 8),), jnp.int32),
            pltpu.VMEM((zrows, Dh), jnp.float32),
            pltpu.VMEM_SHARED((n_pad, Dh), jnp.float32),
            [pltpu.SemaphoreType.DMA for _ in range(NBUF)],
            [pltpu.SemaphoreType.DMA for _ in range(NBUF)],
            [pltpu.SemaphoreType.DMA for _ in range(NBUF)],
            [pltpu.SemaphoreType.DMA for _ in range(NBUF)],
        ],
        compiler_params=pltpu.CompilerParams(use_tc_tiling_on_sc=False),
    )
    def sc_scatter(h_hbm, src_hbm, dst_hbm, out_hbm, sidx, didx, rows, rows_r,
                   sidx_r, didx_r, zbuf, acc, isem, dsem, gsem, ssem):
        c = lax.axis_index("c")
        s = lax.axis_index("s")
        zv = jnp.zeros((LANES,), jnp.float32)

        def zrow(r, carry):
            for q in range(Dh // LANES):
                zbuf[r, pl.ds(q * LANES, LANES)] = zv
            return carry

        lax.fori_loop(0, zrows, zrow, 0)
        r0 = s * rpt
        eoff = s * ept
        for i in range(K):
            for j in range(rpt // zrows):
                pltpu.sync_copy(zbuf, acc.at[pl.ds(r0 + j * zrows, zrows)])
            plsc.subcore_barrier()

            def do_chunks(first_chunk, cnt, sb, db, rb, ism, dsm, gsm):
                off = eoff + first_chunk * C
                icp = pltpu.async_copy(
                    src_hbm.at[c, i, pl.ds(off, cnt)], sb, ism)
                dcp = pltpu.async_copy(
                    dst_hbm.at[i, pl.ds(off, cnt)], db, dsm)
                return icp, dcp

            def burst(t, carry):
                base = t * NBUF
                cps = [
                    do_chunks(base + b, C, sidx[b], didx[b], rows[b],
                              isem[b], dsem[b], gsem[b])
                    for b in range(NBUF)
                ]
                gcps = []
                for b in range(NBUF):
                    cps[b][0].wait()
                    gcps.append(pltpu.async_copy(h_hbm.at[sidx[b]], rows[b], gsem[b]))
                scps = []
                for b in range(NBUF):
                    gcps[b].wait()
                    cps[b][1].wait()
                    scps.append(pltpu.async_copy(rows[b], acc.at[didx[b]],
                                                 ssem[b], add=True))
                for scp in scps:
                    scp.wait()
                return carry

            lax.fori_loop(0, nburst, burst, 0)

            eps = []
            for b in range(nleft):
                eps.append((do_chunks(nburst * NBUF + b, C, sidx[b], didx[b],
                                      rows[b], isem[b], dsem[b], gsem[b]),
                            sidx[b], didx[b], rows[b], gsem[b]))
            if rem:
                off = eoff + nfull * C
                icp = pltpu.async_copy(
                    src_hbm.at[c, i, pl.ds(off, rem)], sidx_r, isem[NBUF - 1])
                dcp = pltpu.async_copy(
                    dst_hbm.at[i, pl.ds(off, rem)], didx_r, dsem[NBUF - 1])
                eps.append(((icp, dcp), sidx_r, didx_r, rows_r, gsem[NBUF - 1]))
            gcps = []
            for (icp, dcp), sb, db, rb, gsm in eps:
                icp.wait()
                gcps.append((pltpu.async_copy(h_hbm.at[sb], rb, gsm), dcp, db, rb))
            scps = []
            for e, (gcp, dcp, db, rb) in enumerate(gcps):
                gcp.wait()
                dcp.wait()
                scps.append(pltpu.async_copy(rb, acc.at[db], ssem[e], add=True))
            for scp in scps:
                scp.wait()

            plsc.subcore_barrier()
            obase = (c * K + i) * n_pad + r0
            pltpu.sync_copy(acc.at[pl.ds(r0, rpt)], out_hbm.at[pl.ds(obase, rpt)])

    return sc_scatter, n_pad


def _fc_body(K, p_ref, w1_ref, b1_ref, w2_ref, b2_ref, w3_ref, b3_ref, o_ref):
    p = p_ref[...]
    halves = []
    for h in range(2):
        a = [jnp.maximum(p[h, i], 0.0) for i in range(K)]
        b = [
            jnp.maximum(
                sum(a[i] * w1_ref[i, j] for i in range(K)) + b1_ref[j], 0.0
            )
            for j in range(K)
        ]
        c = [
            jnp.maximum(
                sum(b[i] * w2_ref[i, j] for i in range(K)) + b2_ref[j], 0.0
            )
            for j in range(K)
        ]
        halves.append(sum(c[i] * w3_ref[i, 0] for i in range(K)) + b3_ref[0])
    o_ref[...] = jnp.concatenate(halves, axis=-1)


def _fc(partial, fc1_w, fc1_b, fc2_w, fc2_b, fc3_w, fc3_b, N, bn):
    _, K, _, Dh = partial.shape
    nb = N // bn
    smem = pl.BlockSpec(memory_space=pltpu.SMEM)
    return pl.pallas_call(
        functools.partial(_fc_body, K),
        grid=(nb,),
        in_specs=[
            pl.BlockSpec((2, K, bn, Dh), lambda n: (0, 0, n, 0)),
            smem, smem, smem, smem, smem, smem,
        ],
        out_specs=pl.BlockSpec((bn, 2 * Dh), lambda n: (n, 0)),
        out_shape=jax.ShapeDtypeStruct((N, 2 * Dh), jnp.float32),
    )(partial, fc1_w, fc1_b, fc2_w, fc2_b, fc3_w, fc3_b)


def kernel(x, edge_index, W, fc1_w, fc1_b, fc2_w, fc2_b, fc3_w, fc3_b):
    N, D_in = x.shape
    K, _, D_out = W.shape
    E = edge_index.shape[2]
    h = _matmul(x, W, bn=2000)
    h_half = h.reshape(K * N * 2, D_out // 2)
    src = edge_index[:, 0, :]
    src_adj = (
        2 * (src[None] + (jnp.arange(K, dtype=jnp.int32) * N)[None, :, None])
        + jnp.arange(NC, dtype=jnp.int32)[:, None, None]
    )
    dst = edge_index[:, 1, :]
    sc, n_pad = _make_sc_scatter(K, N, D_out, E)
    partial = sc(h_half, src_adj, dst)
    partial = partial.reshape(2, K, n_pad, D_out // 2)
    return _fc(partial, fc1_w, fc1_b, fc2_w, fc2_b, fc3_w, fc3_b, N, bn=2000)
